# baseline (speedup 1.0000x reference)
import jax
import jax.numpy as jnp
from jax import lax
from jax.experimental import pallas as pl
from jax.experimental.pallas import tpu as pltpu
from jax.experimental.pallas import tpu_sc as plsc

N = 10000
D = 128
E = 320000

NC = 2
NS = 16
NW = NC * NS
CHUNK = 64
NBUF = 4
NCHUNKS = 160
EDGES_PER_TILE = NCHUNKS * CHUNK
NPAD = NS * 640
ROWS_PER_TILE = NPAD // NS

_MESH = plsc.VectorSubcoreMesh(core_axis_name="c", subcore_axis_name="s")


def _sc_pass(table, src3, dst3, zeros_nd, zeros_n, ones_c, with_deg):
  out_type = [jax.ShapeDtypeStruct((NC, NPAD, D), jnp.float32)]
  scratch = [
      pltpu.VMEM_SHARED((NPAD, D), jnp.float32),
      [pltpu.VMEM((2, CHUNK), jnp.int32) for _ in range(NBUF)],
      [pltpu.VMEM((CHUNK, D), jnp.float32) for _ in range(NBUF)],
      [pltpu.SemaphoreType.DMA for _ in range(NBUF)],
      [pltpu.SemaphoreType.DMA for _ in range(NBUF)],
  ]
  if with_deg:
    out_type.append(jax.ShapeDtypeStruct((NC, NPAD), jnp.float32))
    scratch.append(pltpu.VMEM_SHARED((NPAD,), jnp.float32))
    scratch.append(pltpu.VMEM((CHUNK,), jnp.float32))

  def body(table_hbm, src_hbm, dst_hbm, znd_hbm, zn_hbm, ones_hbm,
           *outs_and_scratch):
    if with_deg:
      (out_h, out_deg, acc, ibufs, rowbufs, isems, gsems,
       deg, ones_v) = outs_and_scratch
    else:
      out_h, acc, ibufs, rowbufs, isems, gsems = outs_and_scratch
    c = lax.axis_index("c")
    s = lax.axis_index("s")
    wid = c * NS + s

    pltpu.sync_copy(znd_hbm.at[pl.ds(s * ROWS_PER_TILE, ROWS_PER_TILE)],
                    acc.at[pl.ds(s * ROWS_PER_TILE, ROWS_PER_TILE)])
    if with_deg:
      pltpu.sync_copy(zn_hbm.at[pl.ds(s * ROWS_PER_TILE, ROWS_PER_TILE)],
                      deg.at[pl.ds(s * ROWS_PER_TILE, ROWS_PER_TILE)])
      pltpu.sync_copy(ones_hbm, ones_v)
    plsc.subcore_barrier()

    def sload(k, b):
      pltpu.async_copy(src_hbm.at[wid, k], ibufs[b], isems[b])

    def swait(b):
      pltpu.make_async_copy(src_hbm.at[wid, 0], ibufs[b], isems[b]).wait()

    def gather(b):
      pltpu.async_copy(table_hbm.at[ibufs[b].at[0]], rowbufs[b], gsems[b])

    def gwait(b):
      pltpu.make_async_copy(table_hbm.at[ibufs[b].at[0]], rowbufs[b],
                            gsems[b]).wait()

    def scatter(k, b):
      del k
      pltpu.sync_copy(rowbufs[b], acc.at[ibufs[b].at[1]], add=True)
      if with_deg:
        pltpu.sync_copy(ones_v, deg.at[ibufs[b].at[1]], add=True)

    for b in range(NBUF):
      sload(b, b)
    for b in range(NBUF):
      swait(b)
      gather(b)

    def step(kk, carry):
      k0 = NBUF * kk
      for b in range(NBUF):
        gwait(b)
        scatter(k0 + b, b)

        @pl.when(kk < NCHUNKS // NBUF - 1)
        def _():
          sload(k0 + NBUF + b, b)
          swait(b)
          gather(b)
      return carry

    lax.fori_loop(0, NCHUNKS // NBUF, step, 0)
    plsc.subcore_barrier()

    pltpu.sync_copy(acc.at[pl.ds(s * ROWS_PER_TILE, ROWS_PER_TILE)],
                    out_h.at[c, pl.ds(s * ROWS_PER_TILE, ROWS_PER_TILE)])
    if with_deg:
      pltpu.sync_copy(deg.at[pl.ds(s * ROWS_PER_TILE, ROWS_PER_TILE)],
                      out_deg.at[c, pl.ds(s * ROWS_PER_TILE, ROWS_PER_TILE)])

  fn = pl.kernel(body, out_type=out_type, mesh=_MESH, scratch_types=scratch)
  return fn(table, src3, dst3, zeros_nd, zeros_n, ones_c)


def _combine_body(pa_ref, pd_ref, out_ref):
  total = pa_ref[0] + pa_ref[1]
  deg = pd_ref[0] + pd_ref[1]
  inv = 1.0 / jnp.maximum(deg, 1.0)
  out_ref[...] = total * inv


_ROWB = 1024


def _combine(pa, pd3):
  grid = (NPAD // _ROWB,)
  return pl.pallas_call(
      _combine_body,
      grid=grid,
      in_specs=[
          pl.BlockSpec((NC, _ROWB, D), lambda i: (0, i, 0)),
          pl.BlockSpec((NC, _ROWB, 1), lambda i: (0, i, 0)),
      ],
      out_specs=pl.BlockSpec((_ROWB, D), lambda i: (i, 0)),
      out_shape=jax.ShapeDtypeStruct((NPAD, D), jnp.float32),
  )(pa, pd3)


def kernel(x, edge_index):
  ei = edge_index.astype(jnp.int32)
  ei3 = ei.reshape(2, NW, E // NW)
  ei3 = jnp.pad(ei3, ((0, 0), (0, 0), (0, EDGES_PER_TILE - E // NW)),
                constant_values=N)
  src3 = jnp.transpose(ei3.reshape(2, NW, NCHUNKS, CHUNK), (1, 2, 0, 3))
  dst3 = jnp.zeros((1,), jnp.int32)
  xp = jnp.pad(x, ((0, NPAD - N), (0, 0)))
  zeros_nd = jnp.zeros((NPAD, D), jnp.float32)
  zeros_n = jnp.zeros((NPAD,), jnp.float32)
  ones_c = jnp.ones((CHUNK,), jnp.float32)

  ph, pdeg = _sc_pass(xp, src3, dst3, zeros_nd, zeros_n, ones_c, with_deg=True)
  pd3 = pdeg[:, :, None]
  h = _combine(ph, pd3)
  (ph2,) = _sc_pass(h, src3, dst3, zeros_nd, zeros_n, ones_c, with_deg=False)
  return _combine(ph2, pd3)[:N]
